# Initial kernel scaffold; baseline (speedup 1.0000x reference)
import jax
import jax.numpy as jnp
from jax.experimental import pallas as pl


def kernel(input, weights):
    raise NotImplementedError("write your pallas kernel here")


# speedup vs baseline: 1.4785x; 1.4785x over previous
import functools

import jax
import jax.numpy as jnp
from jax import lax
from jax.experimental import pallas as pl
from jax.experimental.pallas import tpu as pltpu
from jax.experimental.pallas import tpu_sc as plsc

DIM = 32
BATCH = 4096
HIST = 200
B_TOTAL = BATCH * HIST
NUM_WORKERS = 32
B_PER_W = B_TOTAL // NUM_WORKERS
CHUNK = 1600
N_CHUNKS = B_PER_W // CHUNK


@functools.partial(
    pl.kernel,
    out_type=jax.ShapeDtypeStruct((B_TOTAL, DIM), jnp.float32),
    mesh=plsc.VectorSubcoreMesh(core_axis_name="c", subcore_axis_name="s"),
    scratch_types=[
        pltpu.VMEM((CHUNK,), jnp.int32),
        pltpu.VMEM((CHUNK, DIM), jnp.float32),
        pltpu.SemaphoreType.DMA,
    ],
    compiler_params=pltpu.CompilerParams(use_tc_tiling_on_sc=False),
)
def _embedding_gather(idx_hbm, table_hbm, out_hbm, idx_v, rows_v, sem):
    wid = lax.axis_index("s") * 2 + lax.axis_index("c")
    base = wid * B_PER_W

    def body(i, carry):
        off = base + i * CHUNK
        pltpu.sync_copy(idx_hbm.at[pl.ds(off, CHUNK)], idx_v)
        pltpu.async_copy(table_hbm.at[idx_v], rows_v, sem).wait()
        pltpu.sync_copy(rows_v, out_hbm.at[pl.ds(off, CHUNK)])
        return carry

    lax.fori_loop(0, N_CHUNKS, body, 0, unroll=False)


def kernel(input, weights):
    idx = input.reshape(-1).astype(jnp.int32)
    out = _embedding_gather(idx, weights)
    return out.reshape(BATCH, HIST, DIM)


# speedup vs baseline: 1.4930x; 1.0098x over previous
import functools

import jax
import jax.numpy as jnp
from jax import lax
from jax.experimental import pallas as pl
from jax.experimental.pallas import tpu as pltpu
from jax.experimental.pallas import tpu_sc as plsc

DIM = 32
BATCH = 4096
HIST = 200
B_TOTAL = BATCH * HIST
NUM_WORKERS = 32
B_PER_W = B_TOTAL // NUM_WORKERS
CHUNK = 1600
N_CHUNKS = B_PER_W // CHUNK


@functools.partial(
    pl.kernel,
    out_type=jax.ShapeDtypeStruct((B_TOTAL, DIM), jnp.float32),
    mesh=plsc.VectorSubcoreMesh(core_axis_name="c", subcore_axis_name="s"),
    scratch_types=[
        pltpu.VMEM((B_PER_W,), jnp.int32),
        pltpu.VMEM((CHUNK, DIM), jnp.float32),
        pltpu.VMEM((CHUNK, DIM), jnp.float32),
        pltpu.SemaphoreType.DMA,
        pltpu.SemaphoreType.DMA,
        pltpu.SemaphoreType.DMA,
        pltpu.SemaphoreType.DMA,
    ],
    compiler_params=pltpu.CompilerParams(use_tc_tiling_on_sc=False),
)
def _embedding_gather(idx_hbm, table_hbm, out_hbm, idx_v, rows0, rows1,
                      sg0, sg1, sw0, sw1):
    wid = lax.axis_index("s") * 2 + lax.axis_index("c")
    base = wid * B_PER_W
    rows = (rows0, rows1)
    sg = (sg0, sg1)
    sw = (sw0, sw1)

    pltpu.sync_copy(idx_hbm.at[pl.ds(base, B_PER_W)], idx_v)

    def gather(g, b):
        src = table_hbm.at[idx_v.at[pl.ds(g * CHUNK, CHUNK)]]
        return pltpu.async_copy(src, rows[b], sg[b])

    def writeback(g, b):
        dst = out_hbm.at[pl.ds(base + g * CHUNK, CHUNK)]
        return pltpu.async_copy(rows[b], dst, sw[b])

    gather(0, 0)
    for g in range(N_CHUNKS):
        b = g % 2
        pltpu.make_async_copy(
            table_hbm.at[idx_v.at[pl.ds(g * CHUNK, CHUNK)]], rows[b], sg[b]
        ).wait()
        if g >= 1:
            pltpu.make_async_copy(
                rows[1 - b], out_hbm.at[pl.ds(base + (g - 1) * CHUNK, CHUNK)],
                sw[1 - b]
            ).wait()
        if g + 1 < N_CHUNKS:
            gather(g + 1, 1 - b)
        writeback(g, b)
    bl = (N_CHUNKS - 1) % 2
    pltpu.make_async_copy(
        rows[bl], out_hbm.at[pl.ds(base + (N_CHUNKS - 1) * CHUNK, CHUNK)],
        sw[bl]
    ).wait()


def kernel(input, weights):
    idx = input.reshape(-1).astype(jnp.int32)
    out = _embedding_gather(idx, weights)
    return out.reshape(BATCH, HIST, DIM)


# speedup vs baseline: 1.8422x; 1.2339x over previous
import functools

import jax
import jax.numpy as jnp
from jax import lax
from jax.experimental import pallas as pl
from jax.experimental.pallas import tpu as pltpu
from jax.experimental.pallas import tpu_sc as plsc

VOCAB = 1000000
DIM = 32
BATCH = 4096
HIST = 200
B_TOTAL = BATCH * HIST
NUM_WORKERS = 32
B_PER_W = B_TOTAL // NUM_WORKERS
CHUNK = 1600
N_CHUNKS = B_PER_W // CHUNK

GROUP = 512
KG = 8
TBLK = KG * GROUP
TC_STEPS = -(-VOCAB // TBLK)
N_ROWS = TC_STEPS * TBLK // 4
V_PAD = N_ROWS * 4


@functools.partial(
    pl.kernel,
    out_type=jax.ShapeDtypeStruct((B_TOTAL, DIM), jnp.float32),
    mesh=plsc.VectorSubcoreMesh(core_axis_name="c", subcore_axis_name="s"),
    scratch_types=[
        pltpu.VMEM((B_PER_W,), jnp.int32),
        pltpu.VMEM((CHUNK, DIM), jnp.float32),
        pltpu.VMEM((CHUNK, DIM), jnp.float32),
        pltpu.SemaphoreType.DMA,
        pltpu.SemaphoreType.DMA,
        pltpu.SemaphoreType.DMA,
        pltpu.SemaphoreType.DMA,
    ],
    compiler_params=pltpu.CompilerParams(use_tc_tiling_on_sc=False),
)
def _embedding_gather(idx_hbm, table_hbm, out_hbm, idx_v, rows0, rows1,
                      sg0, sg1, sw0, sw1):
    wid = lax.axis_index("s") * 2 + lax.axis_index("c")
    base = wid * B_PER_W
    rows = (rows0, rows1)
    sg = (sg0, sg1)
    sw = (sw0, sw1)

    pltpu.sync_copy(idx_hbm.at[pl.ds(base, B_PER_W)], idx_v)

    def xform(c, carry):
        v = idx_v[pl.ds(c * 16, 16)]
        g = (v & ~(GROUP - 1)) | ((v & 127) << 2) | ((v >> 7) & 3)
        idx_v[pl.ds(c * 16, 16)] = g
        return carry

    lax.fori_loop(0, B_PER_W // 16, xform, 0, unroll=8)

    def gather(g, b):
        src = table_hbm.at[idx_v.at[pl.ds(g * CHUNK, CHUNK)]]
        return pltpu.async_copy(src, rows[b], sg[b])

    def writeback(g, b):
        dst = out_hbm.at[pl.ds(base + g * CHUNK, CHUNK)]
        return pltpu.async_copy(rows[b], dst, sw[b])

    gather(0, 0)
    for g in range(N_CHUNKS):
        b = g % 2
        pltpu.make_async_copy(
            table_hbm.at[idx_v.at[pl.ds(g * CHUNK, CHUNK)]], rows[b], sg[b]
        ).wait()
        if g >= 1:
            pltpu.make_async_copy(
                rows[1 - b], out_hbm.at[pl.ds(base + (g - 1) * CHUNK, CHUNK)],
                sw[1 - b]
            ).wait()
        if g + 1 < N_CHUNKS:
            gather(g + 1, 1 - b)
        writeback(g, b)
    bl = (N_CHUNKS - 1) % 2
    pltpu.make_async_copy(
        rows[bl], out_hbm.at[pl.ds(base + (N_CHUNKS - 1) * CHUNK, CHUNK)],
        sw[bl]
    ).wait()


@functools.partial(
    pl.pallas_call,
    out_shape=jax.ShapeDtypeStruct((N_ROWS, 128), jnp.float32),
    grid=(TC_STEPS,),
    in_specs=[pl.BlockSpec((DIM, TBLK), lambda i: (0, i))],
    out_specs=pl.BlockSpec((TBLK // 4, 128), lambda i: (i, 0)),
)
def _tc_transpose(wt_ref, out_ref):
    for k in range(KG):
        for a in range(4):
            blk = wt_ref[:, GROUP * k + 128 * a : GROUP * k + 128 * (a + 1)]
            out_ref[128 * k : 128 * (k + 1), DIM * a : DIM * (a + 1)] = blk.T


def kernel(input, weights):
    idx = input.reshape(-1).astype(jnp.int32)
    w_lin = _tc_transpose(weights.T)
    out = _embedding_gather(idx, w_lin.reshape(V_PAD, DIM))
    return out.reshape(BATCH, HIST, DIM)


# speedup vs baseline: 2.0680x; 1.1226x over previous
import functools

import numpy as np

import jax
import jax.numpy as jnp
from jax import lax
from jax.experimental import pallas as pl
from jax.experimental.pallas import tpu as pltpu
from jax.experimental.pallas import tpu_sc as plsc

VOCAB = 1000000
DIM = 32
BATCH = 4096
HIST = 200
B_TOTAL = BATCH * HIST
NUM_WORKERS = 32
B_PER_W = B_TOTAL // NUM_WORKERS
CHUNK = 1600
N_CHUNKS = B_PER_W // CHUNK

GROUP = 512
KG = 8
TBLK = KG * GROUP
TC_STEPS = -(-VOCAB // TBLK)
N_ROWS = TC_STEPS * TBLK // 4
V_PAD = N_ROWS * 4


@functools.partial(
    pl.kernel,
    out_type=jax.ShapeDtypeStruct((B_TOTAL, DIM), jnp.float32),
    mesh=plsc.VectorSubcoreMesh(core_axis_name="c", subcore_axis_name="s"),
    scratch_types=[
        pltpu.VMEM((B_PER_W,), jnp.int32),
        pltpu.VMEM((CHUNK, DIM), jnp.float32),
        pltpu.VMEM((CHUNK, DIM), jnp.float32),
        pltpu.SemaphoreType.DMA,
        pltpu.SemaphoreType.DMA,
        pltpu.SemaphoreType.DMA,
        pltpu.SemaphoreType.DMA,
    ],
    compiler_params=pltpu.CompilerParams(use_tc_tiling_on_sc=False),
)
def _embedding_gather(idx_hbm, table_hbm, out_hbm, idx_v, rows0, rows1,
                      sg0, sg1, sw0, sw1):
    wid = lax.axis_index("s") * 2 + lax.axis_index("c")
    base = wid * B_PER_W
    rows = (rows0, rows1)
    sg = (sg0, sg1)
    sw = (sw0, sw1)

    pltpu.sync_copy(idx_hbm.at[pl.ds(base, B_PER_W)], idx_v)

    def xform(c, carry):
        v = idx_v[pl.ds(c * 16, 16)]
        g = (v & ~(GROUP - 1)) | ((v & 127) << 2) | ((v >> 7) & 3)
        idx_v[pl.ds(c * 16, 16)] = g
        return carry

    lax.fori_loop(0, B_PER_W // 16, xform, 0, unroll=8)

    def gather(g, b):
        src = table_hbm.at[idx_v.at[pl.ds(g * CHUNK, CHUNK)]]
        return pltpu.async_copy(src, rows[b], sg[b])

    def writeback(g, b):
        dst = out_hbm.at[pl.ds(base + g * CHUNK, CHUNK)]
        return pltpu.async_copy(rows[b], dst, sw[b])

    gather(0, 0)
    for g in range(N_CHUNKS):
        b = g % 2
        pltpu.make_async_copy(
            table_hbm.at[idx_v.at[pl.ds(g * CHUNK, CHUNK)]], rows[b], sg[b]
        ).wait()
        if g >= 1:
            pltpu.make_async_copy(
                rows[1 - b], out_hbm.at[pl.ds(base + (g - 1) * CHUNK, CHUNK)],
                sw[1 - b]
            ).wait()
        if g + 1 < N_CHUNKS:
            gather(g + 1, 1 - b)
        writeback(g, b)
    bl = (N_CHUNKS - 1) % 2
    pltpu.make_async_copy(
        rows[bl], out_hbm.at[pl.ds(base + (N_CHUNKS - 1) * CHUNK, CHUNK)],
        sw[bl]
    ).wait()


@functools.partial(
    pl.pallas_call,
    out_shape=jax.ShapeDtypeStruct((N_ROWS, 128), jnp.float32),
    grid=(TC_STEPS,),
    in_specs=[
        pl.BlockSpec((DIM, TBLK), lambda i: (0, i)),
        pl.BlockSpec((128, 128), lambda i: (0, 0)),
    ],
    out_specs=pl.BlockSpec((TBLK // 4, 128), lambda i: (i, 0)),
)
def _tc_transpose(wt_ref, eye_ref, out_ref):
    for k in range(KG):
        xk = jnp.concatenate(
            [wt_ref[:, GROUP * k + 128 * a : GROUP * k + 128 * (a + 1)]
             for a in range(4)],
            axis=0,
        )
        out_ref[128 * k : 128 * (k + 1), :] = lax.dot_general(
            xk, eye_ref[...],
            (((0,), (0,)), ((), ())),
            preferred_element_type=jnp.float32,
        )


def kernel(input, weights):
    idx = input.reshape(-1).astype(jnp.int32)
    w_lin = _tc_transpose(weights.T, jnp.asarray(np.eye(128, dtype=np.float32)))
    out = _embedding_gather(idx, w_lin.reshape(V_PAD, DIM))
    return out.reshape(BATCH, HIST, DIM)
